# baseline (speedup 1.0000x reference)
import jax
import jax.numpy as jnp
from jax import lax
from jax.experimental import pallas as pl
from jax.experimental.pallas import tpu as pltpu
from jax.experimental.pallas import tpu_sc as plsc

E = 128
HID = 128
G4 = 4 * HID
RNN = 256
CTXF = 2 * E + RNN
DEC = 320
NCTX = 51200
NSAMP = 256
NSUB = 204800
PLEN = 9
NODE_PAD = 1024
OUTV = 11000

NC = 2
NS = 16
NW = NC * NS
SUBT = 400
NT = NCTX // SUBT
TPW = NT // NW
K = 128

BLK = 512
NBLK = NCTX // BLK


PE_ROWS = NCTX * PLEN
PE_PW = PE_ROWS // NW
PE_FULL = PE_PW // K
PE_TAIL = PE_PW - PE_FULL * K


def _pe_body(pflat, ntab, out_pe,
             stok_v0, stok_v1, stok_t, rowbuf0, rowbuf1, rowbuf_t,
             gsemA, gsemB, ssemA, ssemB):
  cid = lax.axis_index("c")
  sid = lax.axis_index("s")
  wid = sid * NC + cid

  wbase = wid * PE_PW

  def pe_prep(c, stok_v):
    pltpu.sync_copy(pflat.at[pl.ds(wbase + c * K, K)], stok_v)

  pe_prep(0, stok_v0)
  pltpu.async_copy(ntab.at[stok_v0], rowbuf0, gsemA)

  def pe_pair(p, _):
    c0 = 2 * p
    c1 = c0 + 1
    pltpu.make_async_copy(ntab.at[stok_v0], rowbuf0, gsemA).wait()

    @pl.when(p > 0)
    def _():
      pltpu.make_async_copy(rowbuf1, out_pe.at[pl.ds(0, K)], ssemB).wait()
    pe_prep(c1, stok_v1)
    pltpu.async_copy(ntab.at[stok_v1], rowbuf1, gsemB)
    pltpu.async_copy(rowbuf0, out_pe.at[pl.ds(wbase + c0 * K, K)], ssemA)
    pltpu.make_async_copy(ntab.at[stok_v1], rowbuf1, gsemB).wait()
    pltpu.make_async_copy(rowbuf0, out_pe.at[pl.ds(0, K)], ssemA).wait()

    @pl.when(c0 + 2 < PE_FULL)
    def _():
      pe_prep(c0 + 2, stok_v0)
      pltpu.async_copy(ntab.at[stok_v0], rowbuf0, gsemA)
    pltpu.async_copy(rowbuf1, out_pe.at[pl.ds(wbase + c1 * K, K)], ssemB)
    return 0

  lax.fori_loop(0, PE_FULL // 2, pe_pair, 0)
  pltpu.make_async_copy(rowbuf1, out_pe.at[pl.ds(0, K)], ssemB).wait()
  pltpu.sync_copy(pflat.at[pl.ds(wbase + PE_FULL * K, PE_TAIL)], stok_t)
  pltpu.async_copy(ntab.at[stok_t], rowbuf_t, gsemA).wait()
  pltpu.sync_copy(rowbuf_t, out_pe.at[pl.ds(wbase + PE_FULL * K, PE_TAIL)])


def _emb_body(stok_ll, sidx_ll, stok_rl, sidx_rl, owork_ll, owork_rl, zrows,
              table, out_ll, out_rl,
              offs_v, stok_v0, stok_v1, sidx_v, loc_v0, loc_v1,
              rowbuf0, rowbuf1, acc, gsemA, gsemB, ssemA, ssemB):
  cid = lax.axis_index("c")
  sid = lax.axis_index("s")
  wid = sid * NC + cid
  accs = acc.at[sid]
  for stok_h, sidx_h, owork_h, out_h in (
      (stok_ll, sidx_ll, owork_ll, out_ll),
      (stok_rl, sidx_rl, owork_rl, out_rl),
  ):
    pltpu.sync_copy(owork_h.at[wid], offs_v)
    ovec = offs_v[...]
    for t in range(TPW):
      jlo = ovec[t]
      jhi = ovec[t + 1]
      s0 = (wid * TPW + t) * SUBT
      pltpu.sync_copy(zrows, accs)
      jlo8 = (jlo // 8) * 8
      nch = (jhi - jlo8 + (K - 1)) // K

      def prep(c, stok_v, loc_v, jlo=jlo, jhi=jhi, jlo8=jlo8, s0=s0,
               stok_h=stok_h, sidx_h=sidx_h):
        jbase = jlo8 + c * K
        pltpu.sync_copy(stok_h.at[pl.ds(jbase, K)], stok_v)
        pltpu.sync_copy(sidx_h.at[pl.ds(jbase, K)], sidx_v)
        for q in range(K // 16):
          sl = pl.ds(q * 16, 16)
          jv = jbase + q * 16 + lax.iota(jnp.int32, 16)
          seg = sidx_v[sl]
          valid = (jv >= jlo) & (jv < jhi)
          loc_v[sl] = jnp.where(valid, seg - s0, SUBT)

      @pl.when(nch > 0)
      def _():
        prep(0, stok_v0, loc_v0)
        pltpu.async_copy(table.at[stok_v0], rowbuf0, gsemA)

      def pair(p, _, nch=nch, prep=prep, accs=accs):
        c0 = 2 * p
        c1 = c0 + 1
        pltpu.make_async_copy(table.at[stok_v0], rowbuf0, gsemA).wait()
        pltpu.async_copy(rowbuf0, accs.at[loc_v0], ssemA, add=True)

        @pl.when(c1 < nch)
        def _():
          prep(c1, stok_v1, loc_v1)
          pltpu.async_copy(table.at[stok_v1], rowbuf1, gsemB)

        @pl.when(c0 + 2 < nch)
        def _():
          pltpu.make_async_copy(rowbuf0, accs.at[loc_v0], ssemA).wait()
          prep(c0 + 2, stok_v0, loc_v0)
          pltpu.async_copy(table.at[stok_v0], rowbuf0, gsemA)

        @pl.when(c1 < nch)
        def _():
          pltpu.make_async_copy(table.at[stok_v1], rowbuf1, gsemB).wait()
          pltpu.async_copy(rowbuf1, accs.at[loc_v1], ssemB, add=True)
          pltpu.make_async_copy(rowbuf1, accs.at[loc_v1], ssemB).wait()

        @pl.when(jnp.logical_not(c0 + 2 < nch))
        def _():
          pltpu.make_async_copy(rowbuf0, accs.at[loc_v0], ssemA).wait()

        return 0

      lax.fori_loop(0, (nch + 1) // 2, pair, 0)
      pltpu.sync_copy(accs.at[pl.ds(0, SUBT)], out_h.at[pl.ds(s0, SUBT)])


def _embeddings(ll_subtokens, ll_indices, rl_subtokens, rl_indices,
                subtoken_table, paths, node_table):
  pad = lambda x: jnp.concatenate(
      [x.astype(jnp.int32), jnp.zeros((K,), jnp.int32)])
  bounds = (jnp.arange(NT + 1) * SUBT).astype(jnp.int32)
  idxmat = jnp.arange(NW)[:, None] * TPW + jnp.arange(TPW + 1)[None, :]

  def owork_of(seg_idx):
    offs = jnp.sum(seg_idx[:, None] < bounds[None, :], axis=0,
                   dtype=jnp.int32)
    ow = offs[idxmat]
    return jnp.pad(ow, ((0, 0), (0, 16 - (TPW + 1))))

  owork_ll = owork_of(ll_indices)
  owork_rl = owork_of(rl_indices)
  zrows = jnp.zeros((SUBT + 1, E), jnp.float32)

  mesh = plsc.VectorSubcoreMesh(
      core_axis_name="c", subcore_axis_name="s",
      num_cores=NC, num_subcores=NS)
  run_pe = pl.kernel(
      _pe_body,
      out_type=jax.ShapeDtypeStruct((PE_ROWS, E), jnp.float32),
      mesh=mesh,
      scratch_types=[
          pltpu.VMEM((K,), jnp.int32),
          pltpu.VMEM((K,), jnp.int32),
          pltpu.VMEM((PE_TAIL,), jnp.int32),
          pltpu.VMEM((K, E), jnp.float32),
          pltpu.VMEM((K, E), jnp.float32),
          pltpu.VMEM((PE_TAIL, E), jnp.float32),
          pltpu.SemaphoreType.DMA,
          pltpu.SemaphoreType.DMA,
          pltpu.SemaphoreType.DMA,
          pltpu.SemaphoreType.DMA,
      ],
  )
  pe = run_pe(paths.reshape(-1).astype(jnp.int32), node_table)

  run = pl.kernel(
      _emb_body,
      out_type=(jax.ShapeDtypeStruct((NCTX, E), jnp.float32),
                jax.ShapeDtypeStruct((NCTX, E), jnp.float32)),
      mesh=mesh,
      scratch_types=[
          pltpu.VMEM((16,), jnp.int32),
          pltpu.VMEM((K,), jnp.int32),
          pltpu.VMEM((K,), jnp.int32),
          pltpu.VMEM((K,), jnp.int32),
          pltpu.VMEM((K,), jnp.int32),
          pltpu.VMEM((K,), jnp.int32),
          pltpu.VMEM((K, E), jnp.float32),
          pltpu.VMEM((K, E), jnp.float32),
          pltpu.VMEM_SHARED((NS, SUBT + 1, E), jnp.float32),
          pltpu.SemaphoreType.DMA,
          pltpu.SemaphoreType.DMA,
          pltpu.SemaphoreType.DMA,
          pltpu.SemaphoreType.DMA,
      ],
  )
  ll_emb, rl_emb = run(pad(ll_subtokens), pad(ll_indices),
                       pad(rl_subtokens), pad(rl_indices),
                       owork_ll, owork_rl, zrows, subtoken_table)
  return ll_emb, rl_emb, pe


def _s2a_body(pe_r, wih_f, b_f, wih_b, b_b, h_o):
  pes = pe_r[...].astype(jnp.bfloat16)
  pe = [pes[:, t * E:(t + 1) * E] for t in range(PLEN)]

  def sigm(x):
    return 0.5 + 0.5 * jnp.tanh(0.5 * x)

  def lstm(seq, wcat, b):
    bb = b[...]
    wc = wcat[...]
    h = None
    c = None
    for t, x in enumerate(seq):
      if t == 0:
        g = jnp.dot(x, wc[:E, :], preferred_element_type=jnp.float32) + bb
      else:
        xh = jnp.concatenate([x, h], axis=1)
        g = jnp.dot(xh, wc, preferred_element_type=jnp.float32) + bb
      gi = sigm(g[:, 0 * HID:1 * HID])
      gg = jnp.tanh(g[:, 2 * HID:3 * HID])
      go = sigm(g[:, 3 * HID:4 * HID])
      if t == 0:
        c = gi * gg
      else:
        gf = sigm(g[:, 1 * HID:2 * HID])
        c = gf * c + gi * gg
      h = (go * jnp.tanh(c)).astype(jnp.bfloat16)
    return h

  h_f = lstm(pe, wih_f, b_f)
  h_b = lstm(pe[::-1], wih_b, b_b)
  h_o[...] = jnp.concatenate([h_f, h_b], axis=1)


def _s2b_body(ll, rl, hc, wfc, a2, ctx_o, sc_o, gm_o):
  i = pl.program_id(0)
  ctx_in = jnp.concatenate([ll[...].astype(jnp.bfloat16), hc[...],
                            rl[...].astype(jnp.bfloat16)], axis=1)
  ctx = jnp.tanh(jnp.dot(ctx_in, wfc[...], preferred_element_type=jnp.float32))
  ctx_o[...] = ctx
  s = jnp.dot(ctx, a2[...], preferred_element_type=jnp.float32)
  sc_o[...] = s
  m = jnp.max(s).reshape(1, 1)

  @pl.when(i == 0)
  def _():
    gm_o[...] = m

  @pl.when(i > 0)
  def _():
    gm_o[...] = jnp.maximum(gm_o[...], m)


def _s3_body(ctx_r, sc_r, ind_r, gm_r, v_o, u_acc, s_acc):
  i = pl.program_id(0)

  @pl.when(i == 0)
  def _():
    u_acc[...] = jnp.zeros_like(u_acc)
    s_acc[...] = jnp.zeros_like(s_acc)

  w = jnp.exp(sc_r[...] - gm_r[...])
  ind_row = ind_r[0]
  seg = lax.broadcasted_iota(jnp.int32, (NSAMP, 1), 0)
  oh = (seg == ind_row).astype(jnp.float32)
  u_acc[...] += jnp.dot(oh, ctx_r[...] * w, preferred_element_type=jnp.float32)
  s_acc[...] += jnp.dot(oh, w, preferred_element_type=jnp.float32)

  @pl.when(i == pl.num_programs(0) - 1)
  def _():
    denom = s_acc[...]
    denom = jnp.where(denom > 0, denom, 1.0)
    v_o[...] = u_acc[...] / denom


def _s4_body(v_r, wout_r, bout_r, out_o):
  out_o[...] = lax.dot_general(
      v_r[...], wout_r[...], (((1,), (1,)), ((), ())),
      preferred_element_type=jnp.float32) + bout_r[...]


def _tc_pipeline(ll_emb, rl_emb, pe,
                 Wih_f, Whh_f, bih_f, bhh_f, Wih_b, Whh_b, bih_b, bhh_b,
                 W_fc, a, W_out, b_out, indices):
  full = lambda s: pl.BlockSpec(s, lambda i: (0,) * len(s))
  rows = lambda s: pl.BlockSpec(s, lambda i: (i,) + (0,) * (len(s) - 1))

  hcat = pl.pallas_call(
      _s2a_body,
      grid=(NBLK,),
      in_specs=[
          rows((BLK, PLEN * E)),
          full((E + HID, G4)), full((1, G4)),
          full((E + HID, G4)), full((1, G4)),
      ],
      out_specs=rows((BLK, RNN)),
      out_shape=jax.ShapeDtypeStruct((NCTX, RNN), jnp.bfloat16),
  )(pe.reshape(NCTX, PLEN * E),
    jnp.concatenate([Wih_f.T, Whh_f.T], axis=0).astype(jnp.bfloat16),
    (bih_f + bhh_f).reshape(1, G4),
    jnp.concatenate([Wih_b.T, Whh_b.T], axis=0).astype(jnp.bfloat16),
    (bih_b + bhh_b).reshape(1, G4))

  ctx, scores, gmax = pl.pallas_call(
      _s2b_body,
      grid=(NBLK,),
      in_specs=[
          rows((BLK, E)), rows((BLK, E)), rows((BLK, RNN)),
          full((CTXF, DEC)), full((DEC, 1)),
      ],
      out_specs=[rows((BLK, DEC)), rows((BLK, 1)), full((1, 1))],
      out_shape=[
          jax.ShapeDtypeStruct((NCTX, DEC), jnp.float32),
          jax.ShapeDtypeStruct((NCTX, 1), jnp.float32),
          jax.ShapeDtypeStruct((1, 1), jnp.float32),
      ],
  )(ll_emb, rl_emb, hcat,
    W_fc.T.astype(jnp.bfloat16), a.reshape(DEC, 1))

  ind3 = indices.astype(jnp.int32).reshape(NBLK, 1, BLK)
  v = pl.pallas_call(
      _s3_body,
      grid=(NBLK,),
      in_specs=[
          rows((BLK, DEC)), rows((BLK, 1)), rows((1, 1, BLK)), full((1, 1)),
      ],
      out_specs=pl.BlockSpec((NSAMP, DEC), lambda i: (0, 0)),
      out_shape=jax.ShapeDtypeStruct((NSAMP, DEC), jnp.float32),
      scratch_shapes=[
          pltpu.VMEM((NSAMP, DEC), jnp.float32),
          pltpu.VMEM((NSAMP, 1), jnp.float32),
      ],
  )(ctx, scores, ind3, gmax)

  out = pl.pallas_call(
      _s4_body,
      in_specs=[
          pl.BlockSpec((NSAMP, DEC), lambda: (0, 0)),
          pl.BlockSpec((OUTV, DEC), lambda: (0, 0)),
          pl.BlockSpec((1, OUTV), lambda: (0, 0)),
      ],
      out_specs=pl.BlockSpec((NSAMP, OUTV), lambda: (0, 0)),
      out_shape=jax.ShapeDtypeStruct((NSAMP, OUTV), jnp.float32),
  )(v, W_out, b_out.reshape(1, OUTV))
  return out


def kernel(ll_subtokens, ll_indices, rl_subtokens, rl_indices, paths, indices,
           subtoken_table, node_table,
           Wih_f, Whh_f, bih_f, bhh_f, Wih_b, Whh_b, bih_b, bhh_b,
           W_fc, a, W_out, b_out):
  ll_emb, rl_emb, pe = _embeddings(ll_subtokens, ll_indices,
                                   rl_subtokens, rl_indices, subtoken_table,
                                   paths, node_table)
  return _tc_pipeline(ll_emb, rl_emb, pe,
                      Wih_f, Whh_f, bih_f, bhh_f,
                      Wih_b, Whh_b, bih_b, bhh_b,
                      W_fc, a, W_out, b_out, indices)


# speedup vs baseline: 1.4085x; 1.4085x over previous
import jax
import jax.numpy as jnp
from jax import lax
from jax.experimental import pallas as pl
from jax.experimental.pallas import tpu as pltpu
from jax.experimental.pallas import tpu_sc as plsc

E = 128
HID = 128
G4 = 4 * HID
RNN = 256
CTXF = 2 * E + RNN
DEC = 320
NCTX = 51200
NSAMP = 256
NSUB = 204800
PLEN = 9
NODE_PAD = 1024
OUTV = 11000

NC = 2
NS = 16
NW = NC * NS
SUBT = 400
NT = NCTX // SUBT
TPW = NT // NW
K = 128

BLK = 512
NBLK = NCTX // BLK


def _emb_body(stok_ll, sidx_ll, stok_rl, sidx_rl, owork_ll, owork_rl, zrows,
              table, out_ll, out_rl,
              offs_v, stok_v0, stok_v1, sidx_v, loc_v0, loc_v1,
              rowbuf0, rowbuf1, acc, gsemA, gsemB, ssemA, ssemB):
  cid = lax.axis_index("c")
  sid = lax.axis_index("s")
  wid = sid * NC + cid
  accs = acc.at[sid]
  for stok_h, sidx_h, owork_h, out_h in (
      (stok_ll, sidx_ll, owork_ll, out_ll),
      (stok_rl, sidx_rl, owork_rl, out_rl),
  ):
    pltpu.sync_copy(owork_h.at[wid], offs_v)
    ovec = offs_v[...]
    for t in range(TPW):
      jlo = ovec[t]
      jhi = ovec[t + 1]
      s0 = (wid * TPW + t) * SUBT
      pltpu.sync_copy(zrows, accs)
      jlo8 = (jlo // 8) * 8
      nch = (jhi - jlo8 + (K - 1)) // K

      def prep(c, stok_v, loc_v, jlo=jlo, jhi=jhi, jlo8=jlo8, s0=s0,
               stok_h=stok_h, sidx_h=sidx_h):
        jbase = jlo8 + c * K
        pltpu.sync_copy(stok_h.at[pl.ds(jbase, K)], stok_v)
        pltpu.sync_copy(sidx_h.at[pl.ds(jbase, K)], sidx_v)
        for q in range(K // 16):
          sl = pl.ds(q * 16, 16)
          jv = jbase + q * 16 + lax.iota(jnp.int32, 16)
          seg = sidx_v[sl]
          valid = (jv >= jlo) & (jv < jhi)
          loc_v[sl] = jnp.where(valid, seg - s0, SUBT)

      @pl.when(nch > 0)
      def _():
        prep(0, stok_v0, loc_v0)
        pltpu.async_copy(table.at[stok_v0], rowbuf0, gsemA)

      def pair(p, _, nch=nch, prep=prep, accs=accs):
        c0 = 2 * p
        c1 = c0 + 1
        pltpu.make_async_copy(table.at[stok_v0], rowbuf0, gsemA).wait()
        pltpu.async_copy(rowbuf0, accs.at[loc_v0], ssemA, add=True)

        @pl.when(c1 < nch)
        def _():
          prep(c1, stok_v1, loc_v1)
          pltpu.async_copy(table.at[stok_v1], rowbuf1, gsemB)

        @pl.when(c0 + 2 < nch)
        def _():
          pltpu.make_async_copy(rowbuf0, accs.at[loc_v0], ssemA).wait()
          prep(c0 + 2, stok_v0, loc_v0)
          pltpu.async_copy(table.at[stok_v0], rowbuf0, gsemA)

        @pl.when(c1 < nch)
        def _():
          pltpu.make_async_copy(table.at[stok_v1], rowbuf1, gsemB).wait()
          pltpu.async_copy(rowbuf1, accs.at[loc_v1], ssemB, add=True)
          pltpu.make_async_copy(rowbuf1, accs.at[loc_v1], ssemB).wait()

        @pl.when(jnp.logical_not(c0 + 2 < nch))
        def _():
          pltpu.make_async_copy(rowbuf0, accs.at[loc_v0], ssemA).wait()

        return 0

      lax.fori_loop(0, (nch + 1) // 2, pair, 0)
      pltpu.sync_copy(accs.at[pl.ds(0, SUBT)], out_h.at[pl.ds(s0, SUBT)])


def _embeddings(ll_subtokens, ll_indices, rl_subtokens, rl_indices,
                subtoken_table):
  pad = lambda x: jnp.concatenate(
      [x.astype(jnp.int32), jnp.zeros((K,), jnp.int32)])
  bounds = (jnp.arange(NT + 1) * SUBT).astype(jnp.int32)
  idxmat = jnp.arange(NW)[:, None] * TPW + jnp.arange(TPW + 1)[None, :]

  def owork_of(seg_idx):
    offs = jnp.sum(seg_idx[:, None] < bounds[None, :], axis=0,
                   dtype=jnp.int32)
    ow = offs[idxmat]
    return jnp.pad(ow, ((0, 0), (0, 16 - (TPW + 1))))

  owork_ll = owork_of(ll_indices)
  owork_rl = owork_of(rl_indices)
  zrows = jnp.zeros((SUBT + 1, E), jnp.float32)

  mesh = plsc.VectorSubcoreMesh(
      core_axis_name="c", subcore_axis_name="s",
      num_cores=NC, num_subcores=NS)
  run = pl.kernel(
      _emb_body,
      out_type=(jax.ShapeDtypeStruct((NCTX, E), jnp.float32),
                jax.ShapeDtypeStruct((NCTX, E), jnp.float32)),
      mesh=mesh,
      scratch_types=[
          pltpu.VMEM((16,), jnp.int32),
          pltpu.VMEM((K,), jnp.int32),
          pltpu.VMEM((K,), jnp.int32),
          pltpu.VMEM((K,), jnp.int32),
          pltpu.VMEM((K,), jnp.int32),
          pltpu.VMEM((K,), jnp.int32),
          pltpu.VMEM((K, E), jnp.float32),
          pltpu.VMEM((K, E), jnp.float32),
          pltpu.VMEM_SHARED((NS, SUBT + 1, E), jnp.float32),
          pltpu.SemaphoreType.DMA,
          pltpu.SemaphoreType.DMA,
          pltpu.SemaphoreType.DMA,
          pltpu.SemaphoreType.DMA,
      ],
  )
  return run(pad(ll_subtokens), pad(ll_indices),
             pad(rl_subtokens), pad(rl_indices),
             owork_ll, owork_rl, zrows, subtoken_table)


def _s2a_body(pth, nt, wih_f, b_f, wih_b, b_b, h_o):
  ids = pth[...]
  lane = lax.broadcasted_iota(jnp.int32, (1, NODE_PAD), 1)
  node = nt[...]
  pe = []
  for t in range(PLEN):
    oh = (ids[:, t:t + 1] == lane).astype(jnp.bfloat16)
    pe.append(jnp.dot(oh, node,
                      preferred_element_type=jnp.float32).astype(jnp.bfloat16))

  def sigm(x):
    return 0.5 + 0.5 * jnp.tanh(0.5 * x)

  def lstm(seq, wcat, b):
    bb = b[...]
    wc = wcat[...]
    h = None
    c = None
    for t, x in enumerate(seq):
      if t == 0:
        g = jnp.dot(x, wc[:E, :], preferred_element_type=jnp.float32) + bb
      else:
        xh = jnp.concatenate([x, h], axis=1)
        g = jnp.dot(xh, wc, preferred_element_type=jnp.float32) + bb
      gi = sigm(g[:, 0 * HID:1 * HID])
      gg = jnp.tanh(g[:, 2 * HID:3 * HID])
      go = sigm(g[:, 3 * HID:4 * HID])
      if t == 0:
        c = gi * gg
      else:
        gf = sigm(g[:, 1 * HID:2 * HID])
        c = gf * c + gi * gg
      h = (go * jnp.tanh(c)).astype(jnp.bfloat16)
    return h

  h_f = lstm(pe, wih_f, b_f)
  h_b = lstm(pe[::-1], wih_b, b_b)
  h_o[...] = jnp.concatenate([h_f, h_b], axis=1)


def _s2b3_body(ll, rl, hc, ind_r, wfc, a2, v_o, u_acc, s_acc, m_acc):
  i = pl.program_id(0)

  @pl.when(i == 0)
  def _():
    u_acc[...] = jnp.zeros_like(u_acc)
    s_acc[...] = jnp.zeros_like(s_acc)
    m_acc[...] = jnp.full_like(m_acc, -1e30)

  ctx_in = jnp.concatenate([ll[...].astype(jnp.bfloat16), hc[...],
                            rl[...].astype(jnp.bfloat16)], axis=1)
  ctx = jnp.tanh(jnp.dot(ctx_in, wfc[...], preferred_element_type=jnp.float32))
  s = jnp.dot(ctx, a2[...], preferred_element_type=jnp.float32)
  m_old = m_acc[...]
  m_new = jnp.maximum(m_old, jnp.max(s).reshape(1, 1))
  scale = jnp.exp(m_old - m_new)
  w = jnp.exp(s - m_new)
  ind_row = ind_r[0]
  seg = lax.broadcasted_iota(jnp.int32, (NSAMP, 1), 0)
  oh = (seg == ind_row).astype(jnp.float32)
  u_acc[...] = (u_acc[...] * scale
                + jnp.dot(oh, ctx * w, preferred_element_type=jnp.float32))
  s_acc[...] = (s_acc[...] * scale
                + jnp.dot(oh, w, preferred_element_type=jnp.float32))
  m_acc[...] = m_new

  @pl.when(i == pl.num_programs(0) - 1)
  def _():
    denom = s_acc[...]
    denom = jnp.where(denom > 0, denom, 1.0)
    v_o[...] = u_acc[...] / denom


def _s4_body(v_r, wout_r, bout_r, out_o):
  out_o[...] = lax.dot_general(
      v_r[...], wout_r[...], (((1,), (1,)), ((), ())),
      preferred_element_type=jnp.float32) + bout_r[...]


def _tc_pipeline(ll_emb, rl_emb, paths, node_table,
                 Wih_f, Whh_f, bih_f, bhh_f, Wih_b, Whh_b, bih_b, bhh_b,
                 W_fc, a, W_out, b_out, indices):
  node_pad = jnp.pad(node_table, ((0, NODE_PAD - node_table.shape[0]), (0, 0)))
  full = lambda s: pl.BlockSpec(s, lambda i: (0,) * len(s))
  rows = lambda s: pl.BlockSpec(s, lambda i: (i,) + (0,) * (len(s) - 1))

  hcat = pl.pallas_call(
      _s2a_body,
      grid=(NBLK,),
      in_specs=[
          rows((BLK, PLEN)), full((NODE_PAD, E)),
          full((E + HID, G4)), full((1, G4)),
          full((E + HID, G4)), full((1, G4)),
      ],
      out_specs=rows((BLK, RNN)),
      out_shape=jax.ShapeDtypeStruct((NCTX, RNN), jnp.bfloat16),
  )(paths.astype(jnp.int32), node_pad.astype(jnp.bfloat16),
    jnp.concatenate([Wih_f.T, Whh_f.T], axis=0).astype(jnp.bfloat16),
    (bih_f + bhh_f).reshape(1, G4),
    jnp.concatenate([Wih_b.T, Whh_b.T], axis=0).astype(jnp.bfloat16),
    (bih_b + bhh_b).reshape(1, G4))

  ind3 = indices.astype(jnp.int32).reshape(NBLK, 1, BLK)
  v = pl.pallas_call(
      _s2b3_body,
      grid=(NBLK,),
      in_specs=[
          rows((BLK, E)), rows((BLK, E)), rows((BLK, RNN)),
          rows((1, 1, BLK)),
          full((CTXF, DEC)), full((DEC, 1)),
      ],
      out_specs=pl.BlockSpec((NSAMP, DEC), lambda i: (0, 0)),
      out_shape=jax.ShapeDtypeStruct((NSAMP, DEC), jnp.float32),
      scratch_shapes=[
          pltpu.VMEM((NSAMP, DEC), jnp.float32),
          pltpu.VMEM((NSAMP, 1), jnp.float32),
          pltpu.VMEM((1, 1), jnp.float32),
      ],
  )(ll_emb, rl_emb, hcat, ind3,
    W_fc.T.astype(jnp.bfloat16), a.reshape(DEC, 1))

  out = pl.pallas_call(
      _s4_body,
      in_specs=[
          pl.BlockSpec((NSAMP, DEC), lambda: (0, 0)),
          pl.BlockSpec((OUTV, DEC), lambda: (0, 0)),
          pl.BlockSpec((1, OUTV), lambda: (0, 0)),
      ],
      out_specs=pl.BlockSpec((NSAMP, OUTV), lambda: (0, 0)),
      out_shape=jax.ShapeDtypeStruct((NSAMP, OUTV), jnp.float32),
  )(v, W_out, b_out.reshape(1, OUTV))
  return out


def kernel(ll_subtokens, ll_indices, rl_subtokens, rl_indices, paths, indices,
           subtoken_table, node_table,
           Wih_f, Whh_f, bih_f, bhh_f, Wih_b, Whh_b, bih_b, bhh_b,
           W_fc, a, W_out, b_out):
  ll_emb, rl_emb = _embeddings(ll_subtokens, ll_indices,
                               rl_subtokens, rl_indices, subtoken_table)
  return _tc_pipeline(ll_emb, rl_emb, paths, node_table,
                      Wih_f, Whh_f, bih_f, bhh_f,
                      Wih_b, Whh_b, bih_b, bhh_b,
                      W_fc, a, W_out, b_out, indices)
